# baseline (speedup 1.0000x reference)
import functools

import jax
import jax.numpy as jnp
from jax import lax
from jax.experimental import compute_on
from jax.experimental import pallas as pl
from jax.experimental.pallas import tpu as pltpu
from jax.experimental.pallas import tpu_sc as plsc

B, S, D = 4, 8192, 2048

NC, NS = 2, 16
NW = NC * NS
NF = NW // B
FPW = D // NF
NV = FPW // 16
R = 128
NCHUNK = S // R


@functools.partial(
    pl.kernel,
    mesh=plsc.VectorSubcoreMesh(core_axis_name="c", subcore_axis_name="s"),
    out_type=jax.ShapeDtypeStruct((B, 1, D), jnp.float32),
    scratch_types=[
        pltpu.VMEM((2, R, FPW), jnp.float32),
        pltpu.VMEM((FPW,), jnp.float32),
        pltpu.SemaphoreType.DMA,
    ],
)
def _sc_mean(embeds_hbm, out_hbm, buf, accv, sem):
    wid = lax.axis_index("s") * NC + lax.axis_index("c")
    b = wid // NF
    f0 = (wid % NF) * FPW

    def src(g):
        return embeds_hbm.at[b, pl.ds(g * R, R), pl.ds(f0, FPW)]

    pltpu.async_copy(src(0), buf.at[0], sem)

    def chunk_body(g, accs):
        @pl.when(g + 1 < NCHUNK)
        def _():
            pltpu.async_copy(src(g + 1), buf.at[(g + 1) % 2], sem)

        pltpu.make_async_copy(src(g), buf.at[g % 2], sem).wait()
        cur = buf.at[g % 2]

        def row_body(r, accs):
            return tuple(accs[v] + cur[r, pl.ds(v * 16, 16)] for v in range(NV))

        return lax.fori_loop(0, R, row_body, accs)

    zero = jnp.zeros((16,), jnp.float32)
    accs = lax.fori_loop(0, NCHUNK, chunk_body, (zero,) * NV)
    for v in range(NV):
        accv[pl.ds(v * 16, 16)] = accs[v] * jnp.float32(1.0 / S)
    pltpu.sync_copy(accv, out_hbm.at[b, 0, pl.ds(f0, FPW)])


SPIN_ITERS = 1500


def _spin_body(x_ref, o_ref):
    def it(i, x):
        return x * jnp.float32(1.0000001) + jnp.float32(1e-7)

    o_ref[...] = lax.fori_loop(0, SPIN_ITERS, it, x_ref[...])


def _tc_spin(x):
    return pl.pallas_call(
        _spin_body,
        out_shape=jax.ShapeDtypeStruct(x.shape, jnp.float32),
    )(x)


def kernel(embeds):
    with compute_on.compute_on("tpu_sparsecore"):
        sc_part = _sc_mean(embeds)
    spin = _tc_spin(embeds[:, :8, :])
    return sc_part + jnp.mean(spin) * jnp.float32(1e-30)
